# baseline (speedup 1.0000x reference)
import functools

import jax
import jax.numpy as jnp
from jax.experimental import pallas as pl
from jax.experimental.pallas import tpu as pltpu

CODE_DIM = 256
CODEBOOK_NUM = 8
CODEBOOK_SIZE = 512
SCODEBOOK_ROWS = 1024
B = 8
T = 1024

_DIST_PREC = jax.lax.Precision.DEFAULT


def _rvq_kernel(z_ref, scb_ref, sizei_ref, zq_ref, mapped_ref,
                emb_scr, embt_scr, embsq_scr):
    b = pl.program_id(0)

    @pl.when(b == 0)
    def _init():
        size_col = sizei_ref[...]
        riota = jax.lax.broadcasted_iota(jnp.int32,
                                         (CODEBOOK_SIZE, SCODEBOOK_ROWS), 1)
        osel = (riota == size_col).astype(jnp.float32)
        emb = jax.lax.dot_general(
            osel, scb_ref[...], (((1,), (0,)), ((), ())),
            precision=jax.lax.Precision.HIGHEST,
            preferred_element_type=jnp.float32)
        emb_scr[...] = emb
        embt_scr[...] = jnp.transpose(emb, (1, 0))
        embsq_scr[...] = jnp.sum(emb * emb, axis=1, keepdims=True)

    emb = emb_scr[...]
    emb_sq = embsq_scr[...]
    size_rows = jnp.broadcast_to(
        jnp.transpose(sizei_ref[...], (1, 0)),
        (CODEBOOK_NUM, CODEBOOK_SIZE))

    xs = [z_ref[0], z_ref[1]]
    residuals = list(xs)
    zqs = [jnp.zeros_like(xs[0]), jnp.zeros_like(xs[1])]
    idx_rows = [[], []]
    for q in range(CODEBOOK_NUM):
        for j in range(2):
            m = jax.lax.dot_general(
                emb, residuals[j], (((1,), (0,)), ((), ())),
                precision=_DIST_PREC, preferred_element_type=jnp.float32)
            rsq = jnp.sum(residuals[j] * residuals[j], axis=0,
                          keepdims=True)
            d = (rsq - 2.0 * m) + emb_sq
            idx = jnp.argmin(d, axis=0)
            idx_row = idx[None, :]
            lidx = jnp.broadcast_to(idx_row & 127, (CODE_DIM, T))
            chunk = jnp.broadcast_to(idx_row >> 7, (CODE_DIM, T))
            quant = jnp.take_along_axis(
                embt_scr[:, 0:128], lidx, axis=1,
                mode="promise_in_bounds")
            for c in range(1, 4):
                part = jnp.take_along_axis(
                    embt_scr[:, c * 128:(c + 1) * 128], lidx, axis=1,
                    mode="promise_in_bounds")
                quant = jnp.where(chunk == c, part, quant)
            zqs[j] = zqs[j] + quant
            residuals[j] = residuals[j] - quant
            idx_rows[j].append(idx_row)

    for j in range(2):
        idx_all = jnp.concatenate(idx_rows[j], axis=0)
        lidx = idx_all & 127
        chunk = idx_all >> 7
        mapped = jnp.take_along_axis(
            size_rows[:, 0:128], lidx, axis=1, mode="promise_in_bounds")
        for c in range(1, 4):
            part = jnp.take_along_axis(
                size_rows[:, c * 128:(c + 1) * 128], lidx, axis=1,
                mode="promise_in_bounds")
            mapped = jnp.where(chunk == c, part, mapped)
        mapped_ref[j] = mapped
        zq_ref[j] = xs[j] + (zqs[j] - xs[j])


@functools.partial(jax.jit, static_argnames=())
def kernel(z, scodebook, size):
    sizei = size.reshape(CODEBOOK_SIZE, 1)
    zq_bdt, mapped = pl.pallas_call(
        _rvq_kernel,
        grid=(B // 2,),
        in_specs=[
            pl.BlockSpec((2, CODE_DIM, T), lambda b: (b, 0, 0)),
            pl.BlockSpec((SCODEBOOK_ROWS, CODE_DIM), lambda b: (0, 0)),
            pl.BlockSpec((CODEBOOK_SIZE, 1), lambda b: (0, 0)),
        ],
        out_specs=[
            pl.BlockSpec((2, CODE_DIM, T), lambda b: (b, 0, 0)),
            pl.BlockSpec((2, CODEBOOK_NUM, T), lambda b: (b, 0, 0)),
        ],
        out_shape=[
            jax.ShapeDtypeStruct((B, CODE_DIM, T), jnp.float32),
            jax.ShapeDtypeStruct((B, CODEBOOK_NUM, T), jnp.int32),
        ],
        scratch_shapes=[
            pltpu.VMEM((CODEBOOK_SIZE, CODE_DIM), jnp.float32),
            pltpu.VMEM((CODE_DIM, CODEBOOK_SIZE), jnp.float32),
            pltpu.VMEM((CODEBOOK_SIZE, 1), jnp.float32),
        ],
    )(z, scodebook, sizei)
    zq = jnp.transpose(zq_bdt, (0, 2, 1))
    return zq, jnp.transpose(mapped, (1, 0, 2))
